# baseline (speedup 1.0000x reference)
import functools

import jax
import jax.numpy as jnp
from jax import lax
from jax.experimental import pallas as pl
from jax.experimental.pallas import tpu as pltpu
from jax.experimental.pallas import tpu_sc as plsc

_B, _C, _Z, _H, _W = 2, 4, 64, 128, 128
_ROW = _H * _W
_NBZ = _B * _Z
_NELEM = _B * _C * _Z * _H * _W
_INV = 1.0 / _NELEM

_ZB = 8


def _tc_body(pred_ref, gt_ref, w_ref, out_ref):
    x = pred_ref[...]
    g = gt_ref[...]
    cls = jax.lax.broadcasted_iota(jnp.int32, x.shape, 1)
    sel = g[:, None, :, :, :] == cls
    sp = jax.nn.softplus(x)
    pw = w_ref[...].reshape(1, _C, 1, 1, 1)
    loss = jnp.where(sel, pw * (sp - x), sp)
    part = jnp.sum(loss) * _INV

    @pl.when((pl.program_id(0) == 0) & (pl.program_id(1) == 0))
    def _init():
        out_ref[...] = jnp.zeros_like(out_ref)

    out_ref[...] += part


def _kernel_tc(pred, gt, weights):
    grid = (_B, _Z // _ZB)
    out = pl.pallas_call(
        _tc_body,
        grid=grid,
        in_specs=[
            pl.BlockSpec((1, _C, _ZB, _H, _W), lambda b, z: (b, 0, z, 0, 0)),
            pl.BlockSpec((1, _ZB, _H, _W), lambda b, z: (b, z, 0, 0)),
            pl.BlockSpec((1, _C), lambda b, z: (0, 0)),
        ],
        out_specs=pl.BlockSpec((1, 1), lambda b, z: (0, 0)),
        out_shape=jax.ShapeDtypeStruct((1, 1), jnp.float32),
    )(pred, gt, weights.reshape(1, _C))
    return out[0, 0]


_NC, _NS = 2, 16
_NW = _NC * _NS
_BZ_PER_W = _NBZ // _NW


def _softplus16(x):
    e = jnp.exp(-jnp.abs(x))
    s = e / (e + 2.0)
    s2 = s * s
    l = 2.0 * s * (1.0 + s2 * (1.0 / 3.0 + s2 * (0.2 + s2 * (1.0 / 7.0 + s2 * (1.0 / 9.0)))))
    return jnp.maximum(x, 0.0) + l


def _sc_body(pred_hbm, gt_hbm, w_hbm, out_hbm, pbuf, gbuf, wbuf, obuf):
    wid = lax.axis_index("c") * _NS + lax.axis_index("s")
    pltpu.sync_copy(w_hbm, wbuf)
    wvec = wbuf[...]
    acc = jnp.zeros((16,), jnp.float32)
    for jj in range(_BZ_PER_W):
        j = wid * _BZ_PER_W + jj
        b = j // _Z
        z = j - b * _Z
        pltpu.sync_copy(gt_hbm.at[j], gbuf)
        for c in range(_C):
            r = b * (_C * _Z) + c * _Z + z
            pltpu.sync_copy(pred_hbm.at[r], pbuf)
            pwc = wvec[c]

            def body(i, a, c=c, pwc=pwc):
                x = pbuf[pl.ds(i * 16, 16)]
                g = gbuf[pl.ds(i * 16, 16)]
                sp = _softplus16(x)
                return a + jnp.where(g == c, pwc * (sp - x), sp)

            acc = lax.fori_loop(0, _ROW // 16, body, acc, unroll=8)
    obuf[...] = acc * _INV
    pltpu.sync_copy(obuf, out_hbm.at[wid])


def _kernel_sc(pred, gt, weights):
    p2 = pred.reshape(_B * _C * _Z, _ROW)
    g2 = gt.reshape(_NBZ, _ROW)
    wpad = jnp.pad(weights, (0, 16 - _C))
    mesh = plsc.VectorSubcoreMesh(core_axis_name="c", subcore_axis_name="s")
    call = pl.kernel(
        _sc_body,
        out_type=jax.ShapeDtypeStruct((_NW, 16), jnp.float32),
        mesh=mesh,
        scratch_types=[
            pltpu.VMEM((_ROW,), jnp.float32),
            pltpu.VMEM((_ROW,), jnp.int32),
            pltpu.VMEM((16,), jnp.float32),
            pltpu.VMEM((16,), jnp.float32),
        ],
    )
    return jnp.sum(call(p2, g2, wpad))


def kernel(pred, gt, weights):
    return _kernel_sc(pred, gt, weights)


# speedup vs baseline: 6.1385x; 6.1385x over previous
import functools

import jax
import jax.numpy as jnp
from jax import lax
from jax.experimental import pallas as pl
from jax.experimental.pallas import tpu as pltpu
from jax.experimental.pallas import tpu_sc as plsc

_B, _C, _Z, _H, _W = 2, 4, 64, 128, 128
_ROW = _H * _W
_NBZ = _B * _Z
_NELEM = _B * _C * _Z * _H * _W
_INV = 1.0 / _NELEM

_ZB = 8


_LOG2E = 1.4426950408889634
_LN2 = 0.6931471805599453


def _tc_body(pred_ref, gt_ref, w_ref, out_ref):
    g = gt_ref[...]
    acc = jnp.zeros_like(g, dtype=jnp.float32)
    for c in range(_C):
        t = pred_ref[0, c][None] * _LOG2E
        l = jnp.log2(1.0 + jnp.exp2(t))
        acc += jnp.where(g == c, w_ref[c] * (l - t), l)
    part = jnp.sum(acc) * (_LN2 * _INV)

    @pl.when((pl.program_id(0) == 0) & (pl.program_id(1) == 0))
    def _init():
        out_ref[...] = jnp.zeros_like(out_ref)

    out_ref[...] += part


def _kernel_tc(pred, gt, weights):
    grid = (_B, _Z // _ZB)
    out = pl.pallas_call(
        _tc_body,
        grid=grid,
        in_specs=[
            pl.BlockSpec((1, _C, _ZB, _H, _W), lambda b, z: (b, 0, z, 0, 0)),
            pl.BlockSpec((1, _ZB, _H, _W), lambda b, z: (b, z, 0, 0)),
            pl.BlockSpec(memory_space=pltpu.SMEM),
        ],
        out_specs=pl.BlockSpec((1, 1), lambda b, z: (0, 0)),
        out_shape=jax.ShapeDtypeStruct((1, 1), jnp.float32),
    )(pred, gt, weights)
    return out[0, 0]


_NC, _NS = 2, 16
_NW = _NC * _NS
_BZ_PER_W = _NBZ // _NW


def _softplus16(x):
    e = jnp.exp(-jnp.abs(x))
    s = e / (e + 2.0)
    s2 = s * s
    l = 2.0 * s * (1.0 + s2 * (1.0 / 3.0 + s2 * (0.2 + s2 * (1.0 / 7.0 + s2 * (1.0 / 9.0)))))
    return jnp.maximum(x, 0.0) + l


def _sc_body(pred_hbm, gt_hbm, w_hbm, out_hbm, pbuf, gbuf, wbuf, obuf):
    wid = lax.axis_index("c") * _NS + lax.axis_index("s")
    pltpu.sync_copy(w_hbm, wbuf)
    wvec = wbuf[...]
    acc = jnp.zeros((16,), jnp.float32)
    for jj in range(_BZ_PER_W):
        j = wid * _BZ_PER_W + jj
        b = j // _Z
        z = j - b * _Z
        pltpu.sync_copy(gt_hbm.at[j], gbuf)
        for c in range(_C):
            r = b * (_C * _Z) + c * _Z + z
            pltpu.sync_copy(pred_hbm.at[r], pbuf)
            pwc = wvec[c]

            def body(i, a, c=c, pwc=pwc):
                x = pbuf[pl.ds(i * 16, 16)]
                g = gbuf[pl.ds(i * 16, 16)]
                sp = _softplus16(x)
                return a + jnp.where(g == c, pwc * (sp - x), sp)

            acc = lax.fori_loop(0, _ROW // 16, body, acc, unroll=8)
    obuf[...] = acc * _INV
    pltpu.sync_copy(obuf, out_hbm.at[wid])


def _kernel_sc(pred, gt, weights):
    p2 = pred.reshape(_B * _C * _Z, _ROW)
    g2 = gt.reshape(_NBZ, _ROW)
    wpad = jnp.pad(weights, (0, 16 - _C))
    mesh = plsc.VectorSubcoreMesh(core_axis_name="c", subcore_axis_name="s")
    call = pl.kernel(
        _sc_body,
        out_type=jax.ShapeDtypeStruct((_NW, 16), jnp.float32),
        mesh=mesh,
        scratch_types=[
            pltpu.VMEM((_ROW,), jnp.float32),
            pltpu.VMEM((_ROW,), jnp.int32),
            pltpu.VMEM((16,), jnp.float32),
            pltpu.VMEM((16,), jnp.float32),
        ],
    )
    return jnp.sum(call(p2, g2, wpad))


def kernel(pred, gt, weights):
    return _kernel_tc(pred, gt, weights)
